# baseline (speedup 1.0000x reference)
import functools

import jax
import jax.numpy as jnp
from jax import lax
from jax.experimental import pallas as pl
from jax.experimental.pallas import tpu as pltpu
from jax.experimental.pallas import tpu_sc as plsc

_N = 100000
_NS = 128
_NV = 32
_G = 256
_B = 10000

_NC = 2
_NT = 16
_L = 16
_NW = _NC * _NT
_GP = 272
_C = 3136
_CV = _C // _L


def _tc_body(feats_ref, W1_ref, b1_ref, W2_ref, b2_ref, Wf3_ref,
             forces_ref, e_ref):
    feats = feats_ref[...]
    scal = feats[:, :_NS]
    h = scal @ W1_ref[...] + b1_ref[...]
    h = h * jax.nn.sigmoid(h)
    e_ref[...] = h @ W2_ref[...] + b2_ref[...]
    vecs = feats[:, _NS:]
    forces_ref[...] = vecs @ Wf3_ref[...]


def _sc_body(batch_hbm, ener_hbm, e_out, n_out,
             idx_v, val_v, acc_e, acc_n, red_v, tmp_v, shr_e, shr_n):
    cid = lax.axis_index("c")
    sid = lax.axis_index("s")
    wid = sid * _NC + cid
    base = jnp.minimum(wid * _C, _N - _C)
    valid_from = wid * _C - base

    pltpu.sync_copy(batch_hbm.at[pl.ds(base, _C)], idx_v)
    pltpu.sync_copy(ener_hbm.at[pl.ds(base, _C)], val_v)

    zeros = jnp.zeros((_L,), jnp.float32)
    ones = jnp.ones((_L,), jnp.float32)
    lane = lax.iota(jnp.int32, _L)
    lane_off = lane * _GP

    def zero_col(c, _):
        acc_e[pl.ds(c * _L, _L)] = zeros
        acc_n[pl.ds(c * _L, _L)] = zeros
        return 0
    lax.fori_loop(0, (_L * _GP) // _L, zero_col, 0)

    def scat(j, _):
        b = idx_v[pl.ds(j * _L, _L)]
        e = val_v[pl.ds(j * _L, _L)]
        mask = (j * _L + lane) >= valid_from
        flat = lane_off + b
        plsc.addupdate_scatter(acc_e, [flat], e, mask=mask)
        plsc.addupdate_scatter(acc_n, [flat], ones, mask=mask)
        return 0
    lax.fori_loop(0, _CV, scat, 0)

    def lane_reduce(acc_ref, out1d_ref):
        def red_col(c, _):
            s = acc_ref[pl.ds(c * _L, _L)]
            for r in range(1, _L):
                s = s + acc_ref[pl.ds(r * _GP + c * _L, _L)]
            out1d_ref[pl.ds(c * _L, _L)] = s
            return 0
        lax.fori_loop(0, _GP // _L, red_col, 0)

    lane_reduce(acc_e, red_v)
    pltpu.sync_copy(red_v, shr_e.at[pl.ds(sid * _GP, _GP)])
    lane_reduce(acc_n, red_v)
    pltpu.sync_copy(red_v, shr_n.at[pl.ds(sid * _GP, _GP)])
    plsc.subcore_barrier()

    def tile_reduce(shr, out_hbm):
        pltpu.sync_copy(shr, tmp_v)

        def red_col(c, _):
            s = tmp_v[pl.ds(c * _L, _L)]
            for r in range(1, _NT):
                s = s + tmp_v[pl.ds(r * _GP + c * _L, _L)]
            red_v[pl.ds(c * _L, _L)] = s
            return 0
        lax.fori_loop(0, _G // _L, red_col, 0)
        pltpu.sync_copy(red_v.at[pl.ds(0, _G)], out_hbm.at[cid])

    @pl.when(sid == 0)
    def _():
        tile_reduce(shr_e, e_out)

    @pl.when(sid == 1)
    def _():
        tile_reduce(shr_n, n_out)


@jax.jit
def _sc_segment(batch_i32, energies):
    run = pl.kernel(
        _sc_body,
        mesh=plsc.VectorSubcoreMesh(core_axis_name="c",
                                    subcore_axis_name="s"),
        out_type=[
            jax.ShapeDtypeStruct((_NC, _G), jnp.float32),
            jax.ShapeDtypeStruct((_NC, _G), jnp.float32),
        ],
        scratch_types=[
            pltpu.VMEM((_C,), jnp.int32),
            pltpu.VMEM((_C,), jnp.float32),
            pltpu.VMEM((_L * _GP,), jnp.float32),
            pltpu.VMEM((_L * _GP,), jnp.float32),
            pltpu.VMEM((_GP,), jnp.float32),
            pltpu.VMEM((_NT * _GP,), jnp.float32),
            pltpu.VMEM_SHARED((_NT * _GP,), jnp.float32),
            pltpu.VMEM_SHARED((_NT * _GP,), jnp.float32),
        ],
        compiler_params=pltpu.CompilerParams(needs_layout_passes=False),
    )
    return run(batch_i32, energies)


def kernel(node_feats, batch, W1, b1, W2, b2, Wf):
    n, feat_dim = node_feats.shape
    nsteps = n // _B
    wf3 = (Wf[:, None, None] * jnp.eye(3, dtype=Wf.dtype)).reshape(3 * _NV, 3)

    forces, energies = pl.pallas_call(
        _tc_body,
        grid=(nsteps,),
        in_specs=[
            pl.BlockSpec((_B, feat_dim), lambda i: (i, 0)),
            pl.BlockSpec((_NS, 64), lambda i: (0, 0)),
            pl.BlockSpec((1, 64), lambda i: (0, 0)),
            pl.BlockSpec((64, 1), lambda i: (0, 0)),
            pl.BlockSpec((1, 1), lambda i: (0, 0)),
            pl.BlockSpec((3 * _NV, 3), lambda i: (0, 0)),
        ],
        out_specs=[
            pl.BlockSpec((_B, 3), lambda i: (i, 0)),
            pl.BlockSpec((_B, 1), lambda i: (i, 0)),
        ],
        out_shape=[
            jax.ShapeDtypeStruct((n, 3), jnp.float32),
            jax.ShapeDtypeStruct((n, 1), jnp.float32),
        ],
    )(node_feats, W1, b1.reshape(1, 64), W2, b2.reshape(1, 1), wf3)

    e_parts, n_parts = _sc_segment(batch.astype(jnp.int32),
                                   energies.reshape(n))
    return e_parts.sum(axis=0), forces, n_parts.sum(axis=0)


# speedup vs baseline: 1.1435x; 1.1435x over previous
import functools

import jax
import jax.numpy as jnp
from jax import lax
from jax.experimental import pallas as pl
from jax.experimental.pallas import tpu as pltpu
from jax.experimental.pallas import tpu_sc as plsc

_N = 100000
_NS = 128
_NV = 32
_G = 256
_B = 10000

_NC = 2
_NT = 16
_L = 16
_GP = 272
_C = 3136
_CV = _C // _L


def _tc_body(feats_ref, batch_ref, W1_ref, b1_ref, W2_ref, b2_ref, Wf3_ref,
             forces_ref, acc_ref):
    i = pl.program_id(0)
    feats = feats_ref[...]
    scal = feats[:, :_NS]
    h = scal @ W1_ref[...] + b1_ref[...]
    h = h * jax.nn.sigmoid(h)
    e = h @ W2_ref[...] + b2_ref[...]
    vecs = feats[:, _NS:]
    forces_ref[...] = vecs @ Wf3_ref[...]

    b = batch_ref[0, 0, :]
    oh = (jax.lax.broadcasted_iota(jnp.int32, (_G, _B), 0)
          == b[None, :]).astype(jnp.float32)
    partial = oh @ e

    @pl.when(i == 0)
    def _():
        acc_ref[...] = jnp.zeros_like(acc_ref)
    acc_ref[...] += partial


def _sc_counts_body(batch_hbm, n_out, idx_v, acc_n, red_v, tmp_v, shr_n):
    cid = lax.axis_index("c")
    sid = lax.axis_index("s")
    wid = sid * _NC + cid
    base = jnp.minimum(wid * _C, _N - _C)
    valid_from = wid * _C - base

    pltpu.sync_copy(batch_hbm.at[pl.ds(base, _C)], idx_v)

    zeros = jnp.zeros((_L,), jnp.float32)
    ones = jnp.ones((_L,), jnp.float32)
    lane = lax.iota(jnp.int32, _L)
    lane_off = lane * _GP

    def zero_col(c, _):
        acc_n[pl.ds(c * _L, _L)] = zeros
        return 0
    lax.fori_loop(0, (_L * _GP) // _L, zero_col, 0)

    def scat(j, _):
        b = idx_v[pl.ds(j * _L, _L)]
        mask = (j * _L + lane) >= valid_from
        plsc.addupdate_scatter(acc_n, [lane_off + b], ones, mask=mask)
        return 0
    lax.fori_loop(0, _CV, scat, 0)

    def lane_red_col(c, _):
        s = acc_n[pl.ds(c * _L, _L)]
        for r in range(1, _L):
            s = s + acc_n[pl.ds(r * _GP + c * _L, _L)]
        red_v[pl.ds(c * _L, _L)] = s
        return 0
    lax.fori_loop(0, _GP // _L, lane_red_col, 0)
    pltpu.sync_copy(red_v, shr_n.at[pl.ds(sid * _GP, _GP)])
    plsc.subcore_barrier()

    @pl.when(sid == 0)
    def _():
        pltpu.sync_copy(shr_n, tmp_v)

        def tile_red_col(c, _):
            s = tmp_v[pl.ds(c * _L, _L)]
            for r in range(1, _NT):
                s = s + tmp_v[pl.ds(r * _GP + c * _L, _L)]
            red_v[pl.ds(c * _L, _L)] = s
            return 0
        lax.fori_loop(0, _G // _L, tile_red_col, 0)
        pltpu.sync_copy(red_v.at[pl.ds(0, _G)], n_out.at[cid])


@jax.jit
def _sc_counts(batch_i32):
    run = pl.kernel(
        _sc_counts_body,
        mesh=plsc.VectorSubcoreMesh(core_axis_name="c",
                                    subcore_axis_name="s"),
        out_type=jax.ShapeDtypeStruct((_NC, _G), jnp.float32),
        scratch_types=[
            pltpu.VMEM((_C,), jnp.int32),
            pltpu.VMEM((_L * _GP,), jnp.float32),
            pltpu.VMEM((_GP,), jnp.float32),
            pltpu.VMEM((_NT * _GP,), jnp.float32),
            pltpu.VMEM_SHARED((_NT * _GP,), jnp.float32),
        ],
        compiler_params=pltpu.CompilerParams(needs_layout_passes=False),
    )
    return run(batch_i32)


def kernel(node_feats, batch, W1, b1, W2, b2, Wf):
    n, feat_dim = node_feats.shape
    nsteps = n // _B
    batch32 = batch.astype(jnp.int32)
    n_parts = _sc_counts(batch32)

    wf3 = (Wf[:, None, None] * jnp.eye(3, dtype=Wf.dtype)).reshape(3 * _NV, 3)

    forces, acc = pl.pallas_call(
        _tc_body,
        grid=(nsteps,),
        in_specs=[
            pl.BlockSpec((_B, feat_dim), lambda i: (i, 0)),
            pl.BlockSpec((1, 1, _B), lambda i: (i, 0, 0)),
            pl.BlockSpec((_NS, 64), lambda i: (0, 0)),
            pl.BlockSpec((1, 64), lambda i: (0, 0)),
            pl.BlockSpec((64, 1), lambda i: (0, 0)),
            pl.BlockSpec((1, 1), lambda i: (0, 0)),
            pl.BlockSpec((3 * _NV, 3), lambda i: (0, 0)),
        ],
        out_specs=[
            pl.BlockSpec((_B, 3), lambda i: (i, 0)),
            pl.BlockSpec((_G, 1), lambda i: (0, 0)),
        ],
        out_shape=[
            jax.ShapeDtypeStruct((n, 3), jnp.float32),
            jax.ShapeDtypeStruct((_G, 1), jnp.float32),
        ],
    )(node_feats, batch32.reshape(nsteps, 1, _B), W1, b1.reshape(1, 64),
      W2, b2.reshape(1, 1), wf3)

    return acc[:, 0], forces, n_parts.sum(axis=0)


# speedup vs baseline: 3.2425x; 2.8356x over previous
import functools

import jax
import jax.numpy as jnp
from jax import lax
from jax.experimental import pallas as pl
from jax.experimental.pallas import tpu as pltpu
from jax.experimental.pallas import tpu_sc as plsc

_N = 100000
_NS = 128
_NV = 32
_G = 256
_B = 6400

_NC = 2
_NT = 16
_L = 16
_GP = 272
_C = 3136
_CV = _C // _L


def _tc_body(feats_ref, batch_ref, W1_ref, b1_ref, W2_ref, b2_ref, Wf3_ref,
             forces_ref, acc_ref):
    i = pl.program_id(0)
    feats = feats_ref[...]
    scal = feats[:_NS, :]
    contract00 = (((0,), (0,)), ((), ()))
    h = lax.dot_general(W1_ref[...], scal, contract00) + b1_ref[...]
    h = h * jax.nn.sigmoid(h)
    e = lax.dot_general(W2_ref[...], h, contract00) + b2_ref[...]
    vecs = feats[_NS:, :]
    forces_ref[...] = lax.dot_general(Wf3_ref[...], vecs, contract00)

    col = jax.lax.broadcasted_iota(jnp.int32, (1, _B), 1)
    e = jnp.where(col < (_N - i * _B), e, 0.0)

    b = batch_ref[0, 0, :]
    oh = (jax.lax.broadcasted_iota(jnp.int32, (_G, _B), 0)
          == b[None, :]).astype(jnp.float32)
    partial = lax.dot_general(oh, e, (((1,), (1,)), ((), ())))

    @pl.when(i == 0)
    def _():
        acc_ref[...] = jnp.zeros_like(acc_ref)
    acc_ref[...] += partial


def _sc_counts_body(batch_hbm, n_out, idx_v, acc_n, red_v, tmp_v, shr_n):
    cid = lax.axis_index("c")
    sid = lax.axis_index("s")
    wid = sid * _NC + cid
    base = jnp.minimum(wid * _C, _N - _C)
    valid_from = wid * _C - base

    pltpu.sync_copy(batch_hbm.at[pl.ds(base, _C)], idx_v)

    zeros = jnp.zeros((_L,), jnp.float32)
    ones = jnp.ones((_L,), jnp.float32)
    lane = lax.iota(jnp.int32, _L)
    lane_off = lane * _GP

    def zero_col(c, _):
        acc_n[pl.ds(c * _L, _L)] = zeros
        return 0
    lax.fori_loop(0, (_L * _GP) // _L, zero_col, 0)

    def scat(j, _):
        b = idx_v[pl.ds(j * _L, _L)]
        mask = (j * _L + lane) >= valid_from
        plsc.addupdate_scatter(acc_n, [lane_off + b], ones, mask=mask)
        return 0
    lax.fori_loop(0, _CV, scat, 0)

    def lane_red_col(c, _):
        s = acc_n[pl.ds(c * _L, _L)]
        for r in range(1, _L):
            s = s + acc_n[pl.ds(r * _GP + c * _L, _L)]
        red_v[pl.ds(c * _L, _L)] = s
        return 0
    lax.fori_loop(0, _GP // _L, lane_red_col, 0)
    pltpu.sync_copy(red_v, shr_n.at[pl.ds(sid * _GP, _GP)])
    plsc.subcore_barrier()

    @pl.when(sid == 0)
    def _():
        pltpu.sync_copy(shr_n, tmp_v)

        def tile_red_col(c, _):
            s = tmp_v[pl.ds(c * _L, _L)]
            for r in range(1, _NT):
                s = s + tmp_v[pl.ds(r * _GP + c * _L, _L)]
            red_v[pl.ds(c * _L, _L)] = s
            return 0
        lax.fori_loop(0, _G // _L, tile_red_col, 0)
        pltpu.sync_copy(red_v.at[pl.ds(0, _G)], n_out.at[cid])


@jax.jit
def _sc_counts(batch_i32):
    run = pl.kernel(
        _sc_counts_body,
        mesh=plsc.VectorSubcoreMesh(core_axis_name="c",
                                    subcore_axis_name="s"),
        out_type=jax.ShapeDtypeStruct((_NC, _G), jnp.float32),
        scratch_types=[
            pltpu.VMEM((_C,), jnp.int32),
            pltpu.VMEM((_L * _GP,), jnp.float32),
            pltpu.VMEM((_GP,), jnp.float32),
            pltpu.VMEM((_NT * _GP,), jnp.float32),
            pltpu.VMEM_SHARED((_NT * _GP,), jnp.float32),
        ],
        compiler_params=pltpu.CompilerParams(needs_layout_passes=False),
    )
    return run(batch_i32)


def kernel(node_feats, batch, W1, b1, W2, b2, Wf):
    n, feat_dim = node_feats.shape
    nsteps = (n + _B - 1) // _B
    batch32 = batch.astype(jnp.int32)
    n_parts = _sc_counts(batch32)

    wf3 = (Wf[:, None, None] * jnp.eye(3, dtype=Wf.dtype)).reshape(3 * _NV, 3)

    run = pl.pallas_call(
        _tc_body,
        grid=(nsteps,),
        in_specs=[
            pl.BlockSpec((feat_dim, _B), lambda i: (0, i)),
            pl.BlockSpec((1, 1, _B), lambda i: (i, 0, 0)),
            pl.BlockSpec((_NS, 64), lambda i: (0, 0)),
            pl.BlockSpec((64, 1), lambda i: (0, 0)),
            pl.BlockSpec((64, 1), lambda i: (0, 0)),
            pl.BlockSpec((1, 1), lambda i: (0, 0)),
            pl.BlockSpec((3 * _NV, 3), lambda i: (0, 0)),
        ],
        out_specs=[
            pl.BlockSpec((3, _B), lambda i: (0, i)),
            pl.BlockSpec((_G, 1), lambda i: (0, 0)),
        ],
        out_shape=[
            jax.ShapeDtypeStruct((3, n), jnp.float32),
            jax.ShapeDtypeStruct((_G, 1), jnp.float32),
        ],
    )
    batch_grid = jnp.pad(batch32, (0, nsteps * _B - n),
                         constant_values=2 * _G).reshape(nsteps, 1, _B)
    forces_t, acc = run(node_feats.T, batch_grid, W1, b1.reshape(64, 1),
                        W2, b2.reshape(1, 1), wf3)

    return acc[:, 0], forces_t.T, n_parts.sum(axis=0)


# speedup vs baseline: 3.4096x; 1.0515x over previous
import functools

import jax
import jax.numpy as jnp
from jax import lax
from jax.experimental import pallas as pl
from jax.experimental.pallas import tpu as pltpu
from jax.experimental.pallas import tpu_sc as plsc

_N = 100000
_NS = 128
_NV = 32
_G = 256
_B = 12800

_NC = 2
_NT = 16
_L = 16
_GP = 272
_C = 3136
_CV = _C // _L


def _tc_body(feats_ref, batch_ref, W1_ref, b1_ref, W2_ref, b2_ref, Wf3_ref,
             forces_ref, acc_ref):
    i = pl.program_id(0)
    feats = feats_ref[...]
    scal = feats[:_NS, :]
    contract00 = (((0,), (0,)), ((), ()))
    h = lax.dot_general(W1_ref[...], scal, contract00) + b1_ref[...]
    h = h * jax.nn.sigmoid(h)
    e = lax.dot_general(W2_ref[...], h, contract00) + b2_ref[...]
    vecs = feats[_NS:, :]
    forces_ref[...] = lax.dot_general(Wf3_ref[...], vecs, contract00)

    col = jax.lax.broadcasted_iota(jnp.int32, (1, _B), 1)
    e = jnp.where(col < (_N - i * _B), e, 0.0)

    b = batch_ref[0, 0, :]
    oh = (jax.lax.broadcasted_iota(jnp.int32, (_G, _B), 0)
          == b[None, :]).astype(jnp.float32)
    partial = lax.dot_general(oh, e, (((1,), (1,)), ((), ())))

    @pl.when(i == 0)
    def _():
        acc_ref[...] = jnp.zeros_like(acc_ref)
    acc_ref[...] += partial


def _sc_counts_body(batch_hbm, n_out, idx_v, acc_n, red_v, tmp_v, shr_n):
    cid = lax.axis_index("c")
    sid = lax.axis_index("s")
    wid = sid * _NC + cid
    base = jnp.minimum(wid * _C, _N - _C)
    valid_from = wid * _C - base

    pltpu.sync_copy(batch_hbm.at[pl.ds(base, _C)], idx_v)

    zeros = jnp.zeros((_L,), jnp.float32)
    ones = jnp.ones((_L,), jnp.float32)
    lane = lax.iota(jnp.int32, _L)
    lane_off = lane * _GP

    def zero_col(c, _):
        acc_n[pl.ds(c * _L, _L)] = zeros
        return 0
    lax.fori_loop(0, (_L * _GP) // _L, zero_col, 0)

    def scat(j, _):
        b = idx_v[pl.ds(j * _L, _L)]
        mask = (j * _L + lane) >= valid_from
        plsc.addupdate_scatter(acc_n, [lane_off + b], ones, mask=mask)
        return 0
    lax.fori_loop(0, _CV, scat, 0)

    def lane_red_col(c, _):
        s = acc_n[pl.ds(c * _L, _L)]
        for r in range(1, _L):
            s = s + acc_n[pl.ds(r * _GP + c * _L, _L)]
        red_v[pl.ds(c * _L, _L)] = s
        return 0
    lax.fori_loop(0, _GP // _L, lane_red_col, 0)
    pltpu.sync_copy(red_v, shr_n.at[pl.ds(sid * _GP, _GP)])
    plsc.subcore_barrier()

    @pl.when(sid == 0)
    def _():
        pltpu.sync_copy(shr_n, tmp_v)

        def tile_red_col(c, _):
            s = tmp_v[pl.ds(c * _L, _L)]
            for r in range(1, _NT):
                s = s + tmp_v[pl.ds(r * _GP + c * _L, _L)]
            red_v[pl.ds(c * _L, _L)] = s
            return 0
        lax.fori_loop(0, _G // _L, tile_red_col, 0)
        pltpu.sync_copy(red_v.at[pl.ds(0, _G)], n_out.at[cid])


@jax.jit
def _sc_counts(batch_i32):
    run = pl.kernel(
        _sc_counts_body,
        mesh=plsc.VectorSubcoreMesh(core_axis_name="c",
                                    subcore_axis_name="s"),
        out_type=jax.ShapeDtypeStruct((_NC, _G), jnp.float32),
        scratch_types=[
            pltpu.VMEM((_C,), jnp.int32),
            pltpu.VMEM((_L * _GP,), jnp.float32),
            pltpu.VMEM((_GP,), jnp.float32),
            pltpu.VMEM((_NT * _GP,), jnp.float32),
            pltpu.VMEM_SHARED((_NT * _GP,), jnp.float32),
        ],
        compiler_params=pltpu.CompilerParams(needs_layout_passes=False),
    )
    return run(batch_i32)


def kernel(node_feats, batch, W1, b1, W2, b2, Wf):
    n, feat_dim = node_feats.shape
    nsteps = (n + _B - 1) // _B
    batch32 = batch.astype(jnp.int32)
    n_parts = _sc_counts(batch32)

    wf3 = (Wf[:, None, None] * jnp.eye(3, dtype=Wf.dtype)).reshape(3 * _NV, 3)

    run = pl.pallas_call(
        _tc_body,
        grid=(nsteps,),
        in_specs=[
            pl.BlockSpec((feat_dim, _B), lambda i: (0, i)),
            pl.BlockSpec((1, 1, _B), lambda i: (i, 0, 0)),
            pl.BlockSpec((_NS, 64), lambda i: (0, 0)),
            pl.BlockSpec((64, 1), lambda i: (0, 0)),
            pl.BlockSpec((64, 1), lambda i: (0, 0)),
            pl.BlockSpec((1, 1), lambda i: (0, 0)),
            pl.BlockSpec((3 * _NV, 3), lambda i: (0, 0)),
        ],
        out_specs=[
            pl.BlockSpec((3, _B), lambda i: (0, i)),
            pl.BlockSpec((_G, 1), lambda i: (0, 0)),
        ],
        out_shape=[
            jax.ShapeDtypeStruct((3, n), jnp.float32),
            jax.ShapeDtypeStruct((_G, 1), jnp.float32),
        ],
    )
    batch_grid = jnp.pad(batch32, (0, nsteps * _B - n),
                         constant_values=2 * _G).reshape(nsteps, 1, _B)
    forces_t, acc = run(node_feats.T, batch_grid, W1, b1.reshape(64, 1),
                        W2, b2.reshape(1, 1), wf3)

    return acc[:, 0], forces_t.T, n_parts.sum(axis=0)
